# baseline (speedup 1.0000x reference)
import functools

import jax
import jax.numpy as jnp
from jax import lax
from jax.experimental import pallas as pl
from jax.experimental.pallas import tpu as pltpu
from jax.experimental.pallas import tpu_sc as plsc

_NC = 2
_NS = 16
_NW = _NC * _NS


def _sc_gather(table, idx):
    B, = idx.shape
    V, D = table.shape
    b_per_w = B // _NW
    mesh = plsc.VectorSubcoreMesh(
        core_axis_name="c", subcore_axis_name="s",
        num_cores=_NC, num_subcores=_NS,
    )

    @functools.partial(
        pl.kernel,
        out_type=jax.ShapeDtypeStruct((B, D), table.dtype),
        mesh=mesh,
        scratch_types=[
            pltpu.VMEM((b_per_w,), jnp.int32),
            pltpu.VMEM((b_per_w, D), jnp.float32),
            pltpu.SemaphoreType.DMA,
        ],
    )
    def gather_kernel(table_hbm, idx_hbm, out_hbm, idx_v, rows_v, sem):
        wid = lax.axis_index("s") * _NC + lax.axis_index("c")
        base = wid * b_per_w
        pltpu.sync_copy(idx_hbm.at[pl.ds(base, b_per_w)], idx_v)
        pltpu.async_copy(table_hbm.at[idx_v], rows_v, sem).wait()
        pltpu.sync_copy(rows_v, out_hbm.at[pl.ds(base, b_per_w)])

    return gather_kernel(table, idx)


_NCHUNK = 8


def _tc_dma_concat(emb, feat2):
    B, D = emb.shape
    W = feat2.shape[1]
    K = B // _NCHUNK

    def body(emb_ref, feat_ref, out_ref, esem, fsem):
        copies = [pltpu.make_async_copy(
            emb_ref, out_ref.at[:, pl.ds(0, D)], esem)]
        for c in range(_NCHUNK):
            copies.append(pltpu.make_async_copy(
                feat_ref.at[pl.ds(c * K, K), :],
                out_ref.at[pl.ds(c * K, K), pl.ds(D, W)],
                fsem))
        for cp in copies:
            cp.start()
        for cp in copies:
            cp.wait()

    return pl.pallas_call(
        body,
        in_specs=[
            pl.BlockSpec(memory_space=pltpu.MemorySpace.HBM),
            pl.BlockSpec(memory_space=pltpu.MemorySpace.HBM),
        ],
        out_specs=pl.BlockSpec(memory_space=pltpu.MemorySpace.HBM),
        out_shape=jax.ShapeDtypeStruct((B, D + W), jnp.float32),
        scratch_shapes=[pltpu.SemaphoreType.DMA, pltpu.SemaphoreType.DMA],
    )(emb, feat2)


def kernel(feature, index_value, embedding_table):
    B, T, D = feature.shape
    idx = index_value.astype(jnp.int32)
    emb = _sc_gather(embedding_table, idx)
    out2 = _tc_dma_concat(emb, feature.reshape(B, T * D))
    return out2.reshape(B, T + 1, D)


# speedup vs baseline: 10.3775x; 10.3775x over previous
import functools

import jax
import jax.numpy as jnp
from jax import lax
from jax.experimental import pallas as pl
from jax.experimental.pallas import tpu as pltpu
from jax.experimental.pallas import tpu_sc as plsc

_NC = 2
_NS = 16
_NW = _NC * _NS


def _sc_gather(table, idx):
    B, = idx.shape
    V, D = table.shape
    b_per_w = B // _NW
    mesh = plsc.VectorSubcoreMesh(
        core_axis_name="c", subcore_axis_name="s",
        num_cores=_NC, num_subcores=_NS,
    )

    @functools.partial(
        pl.kernel,
        out_type=jax.ShapeDtypeStruct((B, D), table.dtype),
        mesh=mesh,
        scratch_types=[
            pltpu.VMEM((b_per_w,), jnp.int32),
            pltpu.VMEM((b_per_w, D), jnp.float32),
            pltpu.SemaphoreType.DMA,
        ],
    )
    def gather_kernel(table_hbm, idx_hbm, out_hbm, idx_v, rows_v, sem):
        wid = lax.axis_index("s") * _NC + lax.axis_index("c")
        base = wid * b_per_w
        pltpu.sync_copy(idx_hbm.at[pl.ds(base, b_per_w)], idx_v)
        pltpu.async_copy(table_hbm.at[idx_v], rows_v, sem).wait()
        pltpu.sync_copy(rows_v, out_hbm.at[pl.ds(base, b_per_w)])

    return gather_kernel(table, idx)


_S = 64
_P = 4


def _tc_dma_concat(emb, feat2):
    B, D = emb.shape
    W = feat2.shape[1]
    n_chunks = B // _S
    sp = _S // _P

    def in_copy(feat_ref, fbuf, c, p, sem):
        return pltpu.make_async_copy(
            feat_ref.at[pl.ds(c * _S + p * sp, sp), :],
            fbuf.at[c % 2, pl.ds(p * sp, sp), :],
            sem.at[c % 2])

    def out_copy(out_ref, fbuf, c, p, sem):
        return pltpu.make_async_copy(
            fbuf.at[c % 2, pl.ds(p * sp, sp), :],
            out_ref.at[pl.ds(c * _S + p * sp, sp), pl.ds(D, W)],
            sem.at[c % 2])

    def emb_out_copy(out_ref, ebuf, c, sem):
        return pltpu.make_async_copy(
            ebuf.at[pl.ds(c * _S, _S), :],
            out_ref.at[pl.ds(c * _S, _S), pl.ds(0, D)],
            sem.at[c % 2])

    def body(emb_ref, feat_ref, out_ref, ebuf, fbuf, isem, osem, esem):
        g = pl.program_id(0)

        @pl.when(g == 0)
        def _():
            pltpu.make_async_copy(emb_ref, ebuf, esem.at[0]).start()
            pltpu.make_async_copy(emb_ref, ebuf, esem.at[0]).wait()

        @pl.when(g >= 2)
        def _():
            cc = g - 2
            for p in range(_P):
                out_copy(out_ref, fbuf, cc, p, osem).wait()
            emb_out_copy(out_ref, ebuf, cc, esem).wait()

        @pl.when(g < n_chunks)
        def _():
            for p in range(_P):
                in_copy(feat_ref, fbuf, g, p, isem).start()

        @pl.when((g >= 1) & (g <= n_chunks))
        def _():
            c = g - 1
            for p in range(_P):
                in_copy(feat_ref, fbuf, c, p, isem).wait()
            for p in range(_P):
                out_copy(out_ref, fbuf, c, p, osem).start()
            emb_out_copy(out_ref, ebuf, c, esem).start()

    return pl.pallas_call(
        body,
        grid=(n_chunks + 2,),
        in_specs=[
            pl.BlockSpec(memory_space=pltpu.MemorySpace.HBM),
            pl.BlockSpec(memory_space=pltpu.MemorySpace.HBM),
        ],
        out_specs=pl.BlockSpec(memory_space=pltpu.MemorySpace.HBM),
        out_shape=jax.ShapeDtypeStruct((B, D + W), jnp.float32),
        scratch_shapes=[
            pltpu.VMEM((B, D), jnp.float32),
            pltpu.VMEM((2, _S, W), jnp.float32),
            pltpu.SemaphoreType.DMA((2,)),
            pltpu.SemaphoreType.DMA((2,)),
            pltpu.SemaphoreType.DMA((2,)),
        ],
        compiler_params=pltpu.CompilerParams(
            dimension_semantics=("arbitrary",)),
    )(emb, feat2)


def kernel(feature, index_value, embedding_table):
    B, T, D = feature.shape
    idx = index_value.astype(jnp.int32)
    emb = _sc_gather(embedding_table, idx)
    out2 = _tc_dma_concat(emb, feature.reshape(B, T * D))
    return out2.reshape(B, T + 1, D)


# speedup vs baseline: 10.4763x; 1.0095x over previous
import functools

import jax
import jax.numpy as jnp
from jax import lax
from jax.experimental import pallas as pl
from jax.experimental.pallas import tpu as pltpu
from jax.experimental.pallas import tpu_sc as plsc

_NC = 2
_NS = 16
_NW = _NC * _NS


def _sc_gather(table, idx):
    B, = idx.shape
    V, D = table.shape
    b_per_w = B // _NW
    mesh = plsc.VectorSubcoreMesh(
        core_axis_name="c", subcore_axis_name="s",
        num_cores=_NC, num_subcores=_NS,
    )

    @functools.partial(
        pl.kernel,
        out_type=jax.ShapeDtypeStruct((B, D), table.dtype),
        mesh=mesh,
        scratch_types=[
            pltpu.VMEM((b_per_w,), jnp.int32),
            pltpu.VMEM((b_per_w, D), jnp.float32),
            pltpu.SemaphoreType.DMA,
        ],
    )
    def gather_kernel(table_hbm, idx_hbm, out_hbm, idx_v, rows_v, sem):
        wid = lax.axis_index("s") * _NC + lax.axis_index("c")
        base = wid * b_per_w
        pltpu.sync_copy(idx_hbm.at[pl.ds(base, b_per_w)], idx_v)
        pltpu.async_copy(table_hbm.at[idx_v], rows_v, sem).wait()
        pltpu.sync_copy(rows_v, out_hbm.at[pl.ds(base, b_per_w)])

    return gather_kernel(table, idx)


_S = 64
_P = 4


def _tc_dma_concat(emb, feat2):
    B, D = emb.shape
    W = feat2.shape[1]
    n_chunks = B // _S
    sp = _S // _P

    def in_copy(feat_ref, fbuf, c, p, sem):
        return pltpu.make_async_copy(
            feat_ref.at[pl.ds(c * _S + p * sp, sp), :],
            fbuf.at[c % 4, pl.ds(p * sp, sp), :],
            sem.at[c % 4])

    def out_copy(out_ref, fbuf, c, p, sem):
        return pltpu.make_async_copy(
            fbuf.at[c % 4, pl.ds(p * sp, sp), :],
            out_ref.at[pl.ds(c * _S + p * sp, sp), pl.ds(D, W)],
            sem.at[c % 4])

    def emb_out_copy(out_ref, ebuf, c, sem):
        return pltpu.make_async_copy(
            ebuf.at[pl.ds(c * _S, _S), :],
            out_ref.at[pl.ds(c * _S, _S), pl.ds(0, D)],
            sem.at[c % 4])

    def body(emb_ref, feat_ref, out_ref, ebuf, fbuf, isem, osem, esem):
        g = pl.program_id(0)

        @pl.when(g == 0)
        def _():
            pltpu.make_async_copy(emb_ref, ebuf, esem.at[0]).start()
            pltpu.make_async_copy(emb_ref, ebuf, esem.at[0]).wait()

        @pl.when(g >= 4)
        def _():
            cc = g - 4
            for p in range(_P):
                out_copy(out_ref, fbuf, cc, p, osem).wait()
            emb_out_copy(out_ref, ebuf, cc, esem).wait()

        @pl.when(g < n_chunks)
        def _():
            for p in range(_P):
                in_copy(feat_ref, fbuf, g, p, isem).start()

        @pl.when((g >= 1) & (g <= n_chunks))
        def _():
            c = g - 1
            for p in range(_P):
                in_copy(feat_ref, fbuf, c, p, isem).wait()
            for p in range(_P):
                out_copy(out_ref, fbuf, c, p, osem).start()
            emb_out_copy(out_ref, ebuf, c, esem).start()

    return pl.pallas_call(
        body,
        grid=(n_chunks + 4,),
        in_specs=[
            pl.BlockSpec(memory_space=pltpu.MemorySpace.HBM),
            pl.BlockSpec(memory_space=pltpu.MemorySpace.HBM),
        ],
        out_specs=pl.BlockSpec(memory_space=pltpu.MemorySpace.HBM),
        out_shape=jax.ShapeDtypeStruct((B, D + W), jnp.float32),
        scratch_shapes=[
            pltpu.VMEM((B, D), jnp.float32),
            pltpu.VMEM((4, _S, W), jnp.float32),
            pltpu.SemaphoreType.DMA((4,)),
            pltpu.SemaphoreType.DMA((4,)),
            pltpu.SemaphoreType.DMA((4,)),
        ],
        compiler_params=pltpu.CompilerParams(
            dimension_semantics=("arbitrary",)),
    )(emb, feat2)


def kernel(feature, index_value, embedding_table):
    B, T, D = feature.shape
    idx = index_value.astype(jnp.int32)
    emb = _sc_gather(embedding_table, idx)
    out2 = _tc_dma_concat(emb, feature.reshape(B, T * D))
    return out2.reshape(B, T + 1, D)


# speedup vs baseline: 22.3752x; 2.1358x over previous
import functools

import jax
import jax.numpy as jnp
from jax import lax
from jax.experimental import pallas as pl
from jax.experimental.pallas import tpu as pltpu
from jax.experimental.pallas import tpu_sc as plsc

_NC = 2
_NS = 16
_NW = _NC * _NS


def _sc_gather(table, idx):
    B, = idx.shape
    V, D = table.shape
    b_per_w = B // _NW
    mesh = plsc.VectorSubcoreMesh(
        core_axis_name="c", subcore_axis_name="s",
        num_cores=_NC, num_subcores=_NS,
    )

    @functools.partial(
        pl.kernel,
        out_type=jax.ShapeDtypeStruct((B, D), table.dtype),
        mesh=mesh,
        scratch_types=[
            pltpu.VMEM((b_per_w,), jnp.int32),
            pltpu.VMEM((b_per_w, D), jnp.float32),
            pltpu.SemaphoreType.DMA,
        ],
    )
    def gather_kernel(table_hbm, idx_hbm, out_hbm, idx_v, rows_v, sem):
        wid = lax.axis_index("s") * _NC + lax.axis_index("c")
        base = wid * b_per_w
        pltpu.sync_copy(idx_hbm.at[pl.ds(base, b_per_w)], idx_v)
        pltpu.async_copy(table_hbm.at[idx_v], rows_v, sem).wait()
        pltpu.sync_copy(rows_v, out_hbm.at[pl.ds(base, b_per_w)])

    return gather_kernel(table, idx)


_S = 64
_P = 2
_NSLOT = 3


def _tc_concat(emb, feature):
    B, T, D = feature.shape
    n_chunks = B // _S
    sp = _S // _P

    def in_copy(feat_ref, fbuf, c, p, sem):
        return pltpu.make_async_copy(
            feat_ref.at[pl.ds(c * _S + p * sp, sp)],
            fbuf.at[c % _NSLOT, pl.ds(p * sp, sp)],
            sem.at[c % _NSLOT])

    def out_copy(out_ref, obuf, c, p, sem):
        return pltpu.make_async_copy(
            obuf.at[c % _NSLOT, pl.ds(p * sp, sp)],
            out_ref.at[pl.ds(c * _S + p * sp, sp)],
            sem.at[c % _NSLOT])

    def body(emb_ref, feat_ref, out_ref, ebuf, fbuf, obuf, isem, osem, esem):
        g = pl.program_id(0)

        @pl.when(g == 0)
        def _():
            pltpu.make_async_copy(emb_ref, ebuf, esem).start()
            pltpu.make_async_copy(emb_ref, ebuf, esem).wait()

        @pl.when(g >= 4)
        def _():
            cc = g - 4
            for p in range(_P):
                out_copy(out_ref, obuf, cc, p, osem).wait()

        @pl.when(g < n_chunks)
        def _():
            for p in range(_P):
                in_copy(feat_ref, fbuf, g, p, isem).start()

        @pl.when((g >= 1) & (g <= n_chunks))
        def _():
            c = g - 1
            slot = c % _NSLOT
            for p in range(_P):
                in_copy(feat_ref, fbuf, c, p, isem).wait()
            obuf[slot, :, 0, :] = ebuf[pl.ds(c * _S, _S), :]
            obuf[slot, :, 1:, :] = fbuf[slot]
            for p in range(_P):
                out_copy(out_ref, obuf, c, p, osem).start()

    return pl.pallas_call(
        body,
        grid=(n_chunks + 4,),
        in_specs=[
            pl.BlockSpec(memory_space=pltpu.MemorySpace.HBM),
            pl.BlockSpec(memory_space=pltpu.MemorySpace.HBM),
        ],
        out_specs=pl.BlockSpec(memory_space=pltpu.MemorySpace.HBM),
        out_shape=jax.ShapeDtypeStruct((B, T + 1, D), jnp.float32),
        scratch_shapes=[
            pltpu.VMEM((B, D), jnp.float32),
            pltpu.VMEM((_NSLOT, _S, T, D), jnp.float32),
            pltpu.VMEM((_NSLOT, _S, T + 1, D), jnp.float32),
            pltpu.SemaphoreType.DMA((_NSLOT,)),
            pltpu.SemaphoreType.DMA((_NSLOT,)),
            pltpu.SemaphoreType.DMA,
        ],
        compiler_params=pltpu.CompilerParams(
            dimension_semantics=("arbitrary",)),
    )(emb, feature)


def kernel(feature, index_value, embedding_table):
    idx = index_value.astype(jnp.int32)
    emb = _sc_gather(embedding_table, idx)
    return _tc_concat(emb, feature)


# speedup vs baseline: 22.5114x; 1.0061x over previous
import functools

import jax
import jax.numpy as jnp
from jax import lax
from jax.experimental import pallas as pl
from jax.experimental.pallas import tpu as pltpu
from jax.experimental.pallas import tpu_sc as plsc

_NC = 2
_NS = 16
_NW = _NC * _NS


def _sc_gather(table, idx):
    B, = idx.shape
    V, D = table.shape
    b_per_w = B // _NW
    mesh = plsc.VectorSubcoreMesh(
        core_axis_name="c", subcore_axis_name="s",
        num_cores=_NC, num_subcores=_NS,
    )

    @functools.partial(
        pl.kernel,
        out_type=jax.ShapeDtypeStruct((B, D), table.dtype),
        mesh=mesh,
        scratch_types=[
            pltpu.VMEM((b_per_w,), jnp.int32),
            pltpu.VMEM((b_per_w, D), jnp.float32),
            pltpu.SemaphoreType.DMA,
        ],
    )
    def gather_kernel(table_hbm, idx_hbm, out_hbm, idx_v, rows_v, sem):
        wid = lax.axis_index("s") * _NC + lax.axis_index("c")
        base = wid * b_per_w
        pltpu.sync_copy(idx_hbm.at[pl.ds(base, b_per_w)], idx_v)
        pltpu.async_copy(table_hbm.at[idx_v], rows_v, sem).wait()
        pltpu.sync_copy(rows_v, out_hbm.at[pl.ds(base, b_per_w)])

    return gather_kernel(table, idx)


def _concat_body(emb_ref, feat_ref, out_ref):
    out_ref[:, 0:1, :] = emb_ref[...]
    out_ref[:, 1:, :] = feat_ref[...]


def _tc_concat(emb, feature, block_b=128):
    B, T, D = feature.shape
    emb3 = emb.reshape(B, 1, D)
    return pl.pallas_call(
        _concat_body,
        grid=(B // block_b,),
        in_specs=[
            pl.BlockSpec((block_b, 1, D), lambda b: (b, 0, 0)),
            pl.BlockSpec((block_b, T, D), lambda b: (b, 0, 0)),
        ],
        out_specs=pl.BlockSpec((block_b, T + 1, D), lambda b: (b, 0, 0)),
        out_shape=jax.ShapeDtypeStruct((B, T + 1, D), feature.dtype),
    )(emb3, feature)


def kernel(feature, index_value, embedding_table):
    idx = index_value.astype(jnp.int32)
    emb = _sc_gather(embedding_table, idx)
    return _tc_concat(emb, feature)
